# baseline (speedup 1.0000x reference)
import functools

import jax
import jax.numpy as jnp
from jax import lax
from jax.experimental import pallas as pl
from jax.experimental.pallas import tpu as pltpu
from jax.experimental.pallas import tpu_sc as plsc

N_NODES = 10000
N_EDGES = 320000
D = 128

NC = 2
NS = 16
NW = NC * NS
EPW = N_EDGES // NW
CH = 80
CHUNKS = EPW // CH
RPT = 624
RPT_LAST = N_NODES - (NS - 1) * RPT
ZB = 8

_mesh = plsc.VectorSubcoreMesh(core_axis_name="c", subcore_axis_name="s")

H1 = 5
HC = CHUNKS // H1


@functools.partial(
    pl.kernel,
    out_type=jax.ShapeDtypeStruct((NC, N_NODES, D), jnp.float32),
    mesh=_mesh,
    scratch_types=[
        pltpu.VMEM((HC, CH), jnp.int32),
        pltpu.VMEM((HC, CH), jnp.int32),
        pltpu.VMEM((3, CH, D), jnp.float32),
        pltpu.VMEM((ZB, D), jnp.float32),
        pltpu.VMEM_SHARED((N_NODES, D), jnp.float32),
        pltpu.SemaphoreType.DMA,
        pltpu.SemaphoreType.DMA,
        pltpu.SemaphoreType.DMA,
        pltpu.SemaphoreType.DMA,
        pltpu.SemaphoreType.DMA,
        pltpu.SemaphoreType.DMA,
    ],
)
def _segment_sum_sc(x_hbm, src_hbm, dst_hbm, out_hbm,
                    sidx, didx, bufx, zbuf, acc,
                    sg0, sg1, sg2, ss0, ss1, ss2):
    cid = lax.axis_index("c")
    sid = lax.axis_index("s")
    wid = sid * NC + cid
    sg = (sg0, sg1, sg2)
    ss = (ss0, ss1, ss2)

    @pl.loop(0, ZB)
    def _(i):
        for j in range(D // 16):
            zbuf.at[pl.ds(i, 1), pl.ds(j * 16, 16)][...] = jnp.zeros(
                (1, 16), jnp.float32)

    @pl.when(sid < NS - 1)
    def _():
        @pl.loop(0, RPT // ZB)
        def _(k):
            pltpu.sync_copy(zbuf, acc.at[pl.ds(sid * RPT + k * ZB, ZB)])

    @pl.when(sid == NS - 1)
    def _():
        @pl.loop(0, RPT_LAST // ZB)
        def _(k):
            pltpu.sync_copy(zbuf, acc.at[pl.ds((NS - 1) * RPT + k * ZB, ZB)])

    plsc.subcore_barrier()

    def issue_gather(c, b):
        pltpu.async_copy(x_hbm.at[sidx.at[c]], bufx.at[b], sg[b])

    def wait_gather(b):
        pltpu.make_async_copy(x_hbm.at[sidx.at[0]], bufx.at[b], sg[b]).wait()

    def issue_scatter(c, b):
        pltpu.async_copy(bufx.at[b], acc.at[didx.at[c]], ss[b], add=True)

    def wait_scatter(b):
        pltpu.make_async_copy(bufx.at[b], acc.at[didx.at[0]], ss[b]).wait()

    for h in range(H1):
        pltpu.sync_copy(src_hbm.at[wid, h], sidx)
        pltpu.sync_copy(dst_hbm.at[wid, h], didx)
        issue_gather(0, 0)
        issue_gather(1, 1)
        wait_gather(0)
        issue_scatter(0, 0)
        issue_gather(2, 2)

        @pl.loop(0, (HC - 4) // 3)
        def _(p):
            for k in range(3):
                b = (1 + k) % 3
                c = 1 + p * 3 + k
                wait_gather(b)
                issue_scatter(c, b)
                wait_scatter(k)
                issue_gather(c + 2, k)

        wait_gather((HC - 3) % 3)
        issue_scatter(HC - 3, (HC - 3) % 3)
        wait_scatter((HC - 4) % 3)
        issue_gather(HC - 1, (HC - 1) % 3)
        for c in (HC - 2, HC - 1):
            b = c % 3
            wait_gather(b)
            issue_scatter(c, b)
            wait_scatter((c - 1) % 3)
        wait_scatter((HC - 1) % 3)

    plsc.subcore_barrier()

    @pl.when(sid < NS - 1)
    def _():
        r = sid * RPT
        pltpu.sync_copy(acc.at[pl.ds(r, RPT)], out_hbm.at[cid, pl.ds(r, RPT)])

    @pl.when(sid == NS - 1)
    def _():
        r = (NS - 1) * RPT
        pltpu.sync_copy(acc.at[pl.ds(r, RPT_LAST)],
                        out_hbm.at[cid, pl.ds(r, RPT_LAST)])


_ROWS_BLK = 1000


def _linear_a_body(x_ref, w_ref, b_ref, a_ref):
    a_ref[...] = jnp.dot(x_ref[...], w_ref[...],
                         preferred_element_type=jnp.float32) + b_ref[...]


def _linear_a_tc(x, w1, bb):
    grid = (N_NODES // _ROWS_BLK,)
    blk = pl.BlockSpec((_ROWS_BLK, D), lambda i: (i, 0))
    return pl.pallas_call(
        _linear_a_body,
        grid=grid,
        in_specs=[blk,
                  pl.BlockSpec((D, D), lambda i: (0, 0)),
                  pl.BlockSpec((1, D), lambda i: (0, 0))],
        out_specs=blk,
        out_shape=jax.ShapeDtypeStruct((N_NODES, D), jnp.float32),
    )(x, w1, bb)


def _linear_b_body(s0_ref, s1_ref, w_ref, bb_ref):
    bb_ref[...] = jnp.dot(s0_ref[...] + s1_ref[...], w_ref[...],
                          preferred_element_type=jnp.float32)


def _linear_b_tc(s0, s1, w2):
    grid = (N_NODES // _ROWS_BLK,)
    blk = pl.BlockSpec((_ROWS_BLK, D), lambda i: (i, 0))
    return pl.pallas_call(
        _linear_b_body,
        grid=grid,
        in_specs=[blk, blk, pl.BlockSpec((D, D), lambda i: (0, 0))],
        out_specs=blk,
        out_shape=jax.ShapeDtypeStruct((N_NODES, D), jnp.float32),
    )(s0, s1, w2)


def _linear_body(x_ref, s0_ref, s1_ref, w_ref, b_ref, a_ref, bb_ref):
    w1 = w_ref[0:D, :]
    w2 = w_ref[D:2 * D, :]
    a_ref[...] = jnp.dot(x_ref[...], w1,
                         preferred_element_type=jnp.float32) + b_ref[...]
    bb_ref[...] = jnp.dot(s0_ref[...] + s1_ref[...], w2,
                          preferred_element_type=jnp.float32)


def _linear_tc(x, s0, s1, W, bvec):
    grid = (N_NODES // _ROWS_BLK,)
    blk = pl.BlockSpec((_ROWS_BLK, D), lambda i: (i, 0))
    return pl.pallas_call(
        _linear_body,
        grid=grid,
        in_specs=[blk, blk, blk,
                  pl.BlockSpec((2 * D, D), lambda i: (0, 0)),
                  pl.BlockSpec((1, D), lambda i: (0, 0))],
        out_specs=[blk, blk],
        out_shape=[jax.ShapeDtypeStruct((N_NODES, D), jnp.float32)] * 2,
    )(x, s0, s1, W, bvec.reshape(1, D))


@functools.partial(
    pl.kernel,
    out_type=jax.ShapeDtypeStruct((N_EDGES, D), jnp.float32),
    mesh=_mesh,
    scratch_types=[
        pltpu.VMEM((EPW,), jnp.int32),
        pltpu.VMEM((EPW,), jnp.int32),
        pltpu.VMEM((2, CH, D), jnp.float32),
        pltpu.VMEM((2, CH, D), jnp.float32),
        pltpu.VMEM((2, CH, D), jnp.float32),
        pltpu.SemaphoreType.DMA,
        pltpu.SemaphoreType.DMA,
        pltpu.SemaphoreType.DMA,
        pltpu.SemaphoreType.DMA,
    ],
)
def _edge_combine_sc(a_hbm, b_hbm, src_hbm, dst_hbm, out_hbm,
                     sidx, didx, bufa, bufb, bufo, sg0, sg1, so0, so1):
    cid = lax.axis_index("c")
    sid = lax.axis_index("s")
    wid = sid * NC + cid
    ebase = wid * EPW
    sg = (sg0, sg1)
    so = (so0, so1)

    pltpu.sync_copy(src_hbm.at[pl.ds(ebase, EPW)], sidx)
    pltpu.sync_copy(dst_hbm.at[pl.ds(ebase, EPW)], didx)

    def issue_gathers(c, b):
        pltpu.async_copy(a_hbm.at[sidx.at[pl.ds(c * CH, CH)]], bufa.at[b],
                         sg[b])
        pltpu.async_copy(b_hbm.at[didx.at[pl.ds(c * CH, CH)]], bufb.at[b],
                         sg[b])

    def wait_gathers(b):
        pltpu.make_async_copy(a_hbm.at[sidx.at[pl.ds(0, CH)]], bufa.at[b],
                              sg[b]).wait()
        pltpu.make_async_copy(b_hbm.at[didx.at[pl.ds(0, CH)]], bufb.at[b],
                              sg[b]).wait()

    def do_add(b):
        @pl.loop(0, CH)
        def _(r):
            for j in range(D // 16):
                sl = (b, pl.ds(r, 1), pl.ds(j * 16, 16))
                bufo.at[sl][...] = bufa.at[sl][...] + bufb.at[sl][...]

    def issue_out(c, b):
        pltpu.async_copy(bufo.at[b], out_hbm.at[pl.ds(ebase + c * CH, CH)],
                         so[b])

    def wait_out(b):
        pltpu.make_async_copy(bufo.at[b], out_hbm.at[pl.ds(ebase, CH)],
                              so[b]).wait()

    issue_gathers(0, 0)
    issue_gathers(1, 1)

    @pl.loop(0, (CHUNKS - 1) // 2)
    def _(p):
        for b in range(2):
            c = p * 2 + b
            wait_gathers(b)

            @pl.when(p > 0)
            def _():
                wait_out(b)

            do_add(b)
            if b == 0:
                issue_gathers(c + 2, b)
            else:
                @pl.when(p < (CHUNKS - 1) // 2 - 1)
                def _():
                    issue_gathers(c + 2, b)
            issue_out(c, b)

    wait_gathers(0)
    wait_out(0)
    do_add(0)
    issue_out(CHUNKS - 1, 0)
    wait_out(1)
    wait_out(0)


def kernel(x, edge_index, W, b):
    src = edge_index[0].astype(jnp.int32)
    dst = edge_index[1].astype(jnp.int32)
    s_part = _segment_sum_sc(x, src.reshape(NW, H1, HC, CH),
                             dst.reshape(NW, H1, HC, CH))
    a, bb = _linear_tc(x, s_part[0], s_part[1], W, b)
    return _edge_combine_sc(a, bb, src, dst)


# speedup vs baseline: 1.0264x; 1.0264x over previous
import functools

import jax
import jax.numpy as jnp
from jax import lax
from jax.experimental import pallas as pl
from jax.experimental.pallas import tpu as pltpu
from jax.experimental.pallas import tpu_sc as plsc

N_NODES = 10000
N_EDGES = 320000
D = 128

NC = 2
NS = 16
NW = NC * NS
EPW = N_EDGES // NW
CH = 80
CHUNKS = EPW // CH
RPT = 624
RPT_LAST = N_NODES - (NS - 1) * RPT
ZB = 8

_mesh = plsc.VectorSubcoreMesh(core_axis_name="c", subcore_axis_name="s")

H1 = 5
HC = CHUNKS // H1


@functools.partial(
    pl.kernel,
    out_type=jax.ShapeDtypeStruct((NC, N_NODES, D), jnp.float32),
    mesh=_mesh,
    scratch_types=[
        pltpu.VMEM((HC, CH), jnp.int32),
        pltpu.VMEM((HC, CH), jnp.int32),
        pltpu.VMEM((3, CH, D), jnp.float32),
        pltpu.VMEM((ZB, D), jnp.float32),
        pltpu.VMEM_SHARED((N_NODES, D), jnp.float32),
        pltpu.SemaphoreType.DMA,
        pltpu.SemaphoreType.DMA,
        pltpu.SemaphoreType.DMA,
        pltpu.SemaphoreType.DMA,
        pltpu.SemaphoreType.DMA,
        pltpu.SemaphoreType.DMA,
    ],
)
def _segment_sum_sc(x_hbm, src_hbm, dst_hbm, out_hbm,
                    sidx, didx, bufx, zbuf, acc,
                    sg0, sg1, sg2, ss0, ss1, ss2):
    cid = lax.axis_index("c")
    sid = lax.axis_index("s")
    wid = sid * NC + cid
    sg = (sg0, sg1, sg2)
    ss = (ss0, ss1, ss2)

    @pl.loop(0, ZB)
    def _(i):
        for j in range(D // 16):
            zbuf.at[pl.ds(i, 1), pl.ds(j * 16, 16)][...] = jnp.zeros(
                (1, 16), jnp.float32)

    @pl.when(sid < NS - 1)
    def _():
        @pl.loop(0, RPT // ZB)
        def _(k):
            pltpu.sync_copy(zbuf, acc.at[pl.ds(sid * RPT + k * ZB, ZB)])

    @pl.when(sid == NS - 1)
    def _():
        @pl.loop(0, RPT_LAST // ZB)
        def _(k):
            pltpu.sync_copy(zbuf, acc.at[pl.ds((NS - 1) * RPT + k * ZB, ZB)])

    plsc.subcore_barrier()

    def issue_gather(c, b):
        pltpu.async_copy(x_hbm.at[sidx.at[c]], bufx.at[b], sg[b])

    def wait_gather(b):
        pltpu.make_async_copy(x_hbm.at[sidx.at[0]], bufx.at[b], sg[b]).wait()

    def issue_scatter(c, b):
        pltpu.async_copy(bufx.at[b], acc.at[didx.at[c]], ss[b], add=True)

    def wait_scatter(b):
        pltpu.make_async_copy(bufx.at[b], acc.at[didx.at[0]], ss[b]).wait()

    for h in range(H1):
        pltpu.sync_copy(src_hbm.at[wid, h], sidx)
        pltpu.sync_copy(dst_hbm.at[wid, h], didx)
        issue_gather(0, 0)
        issue_gather(1, 1)
        wait_gather(0)
        issue_scatter(0, 0)
        issue_gather(2, 2)

        @pl.loop(0, (HC - 4) // 3)
        def _(p):
            for k in range(3):
                b = (1 + k) % 3
                c = 1 + p * 3 + k
                wait_gather(b)
                issue_scatter(c, b)
                wait_scatter(k)
                issue_gather(c + 2, k)

        wait_gather((HC - 3) % 3)
        issue_scatter(HC - 3, (HC - 3) % 3)
        wait_scatter((HC - 4) % 3)
        issue_gather(HC - 1, (HC - 1) % 3)
        for c in (HC - 2, HC - 1):
            b = c % 3
            wait_gather(b)
            issue_scatter(c, b)
            wait_scatter((c - 1) % 3)
        wait_scatter((HC - 1) % 3)

    plsc.subcore_barrier()

    @pl.when(sid < NS - 1)
    def _():
        r = sid * RPT
        pltpu.sync_copy(acc.at[pl.ds(r, RPT)], out_hbm.at[cid, pl.ds(r, RPT)])

    @pl.when(sid == NS - 1)
    def _():
        r = (NS - 1) * RPT
        pltpu.sync_copy(acc.at[pl.ds(r, RPT_LAST)],
                        out_hbm.at[cid, pl.ds(r, RPT_LAST)])


_ROWS_BLK = 1000


def _linear_a_body(x_ref, w_ref, b_ref, a_ref):
    a_ref[...] = jnp.dot(x_ref[...], w_ref[...],
                         preferred_element_type=jnp.float32) + b_ref[...]


def _linear_a_tc(x, w1, bb):
    grid = (N_NODES // _ROWS_BLK,)
    blk = pl.BlockSpec((_ROWS_BLK, D), lambda i: (i, 0))
    return pl.pallas_call(
        _linear_a_body,
        grid=grid,
        in_specs=[blk,
                  pl.BlockSpec((D, D), lambda i: (0, 0)),
                  pl.BlockSpec((1, D), lambda i: (0, 0))],
        out_specs=blk,
        out_shape=jax.ShapeDtypeStruct((N_NODES, D), jnp.float32),
    )(x, w1, bb)


def _linear_b_body(s0_ref, s1_ref, w_ref, bb_ref):
    bb_ref[...] = jnp.dot(s0_ref[...] + s1_ref[...], w_ref[...],
                          preferred_element_type=jnp.float32)


def _linear_b_tc(s0, s1, w2):
    grid = (N_NODES // _ROWS_BLK,)
    blk = pl.BlockSpec((_ROWS_BLK, D), lambda i: (i, 0))
    return pl.pallas_call(
        _linear_b_body,
        grid=grid,
        in_specs=[blk, blk, pl.BlockSpec((D, D), lambda i: (0, 0))],
        out_specs=blk,
        out_shape=jax.ShapeDtypeStruct((N_NODES, D), jnp.float32),
    )(s0, s1, w2)


def _linear_body(x_ref, s0_ref, s1_ref, w_ref, b_ref, a_ref, bb_ref):
    w1 = w_ref[0:D, :]
    w2 = w_ref[D:2 * D, :]
    a_ref[...] = jnp.dot(x_ref[...], w1,
                         preferred_element_type=jnp.float32) + b_ref[...]
    bb_ref[...] = jnp.dot(s0_ref[...] + s1_ref[...], w2,
                          preferred_element_type=jnp.float32)


def _linear_tc(x, s0, s1, W, bvec):
    grid = (N_NODES // _ROWS_BLK,)
    blk = pl.BlockSpec((_ROWS_BLK, D), lambda i: (i, 0))
    return pl.pallas_call(
        _linear_body,
        grid=grid,
        in_specs=[blk, blk, blk,
                  pl.BlockSpec((2 * D, D), lambda i: (0, 0)),
                  pl.BlockSpec((1, D), lambda i: (0, 0))],
        out_specs=[blk, blk],
        out_shape=[jax.ShapeDtypeStruct((N_NODES, D), jnp.float32)] * 2,
    )(x, s0, s1, W, bvec.reshape(1, D))


CH3 = 120
F3 = EPW // CH3
T3 = EPW - F3 * CH3


@functools.partial(
    pl.kernel,
    out_type=jax.ShapeDtypeStruct((N_EDGES, D), jnp.float32),
    mesh=_mesh,
    scratch_types=[
        pltpu.VMEM((EPW,), jnp.int32),
        pltpu.VMEM((EPW,), jnp.int32),
        pltpu.VMEM((2, CH3, D), jnp.float32),
        pltpu.VMEM((2, CH3, D), jnp.float32),
        pltpu.VMEM((2, CH3, D), jnp.float32),
        pltpu.SemaphoreType.DMA,
        pltpu.SemaphoreType.DMA,
        pltpu.SemaphoreType.DMA,
        pltpu.SemaphoreType.DMA,
    ],
)
def _edge_combine_sc(a_hbm, b_hbm, src_hbm, dst_hbm, out_hbm,
                     sidx, didx, bufa, bufb, bufo, sg0, sg1, so0, so1):
    cid = lax.axis_index("c")
    sid = lax.axis_index("s")
    wid = sid * NC + cid
    ebase = wid * EPW
    sg = (sg0, sg1)
    so = (so0, so1)

    pltpu.sync_copy(src_hbm.at[pl.ds(ebase, EPW)], sidx)
    pltpu.sync_copy(dst_hbm.at[pl.ds(ebase, EPW)], didx)

    def issue_gathers(c, b, n=CH3):
        pltpu.async_copy(a_hbm.at[sidx.at[pl.ds(c * CH3, n)]],
                         bufa.at[b, pl.ds(0, n)], sg[b])
        pltpu.async_copy(b_hbm.at[didx.at[pl.ds(c * CH3, n)]],
                         bufb.at[b, pl.ds(0, n)], sg[b])

    def wait_gathers(b, n=CH3):
        pltpu.make_async_copy(a_hbm.at[sidx.at[pl.ds(0, n)]],
                              bufa.at[b, pl.ds(0, n)], sg[b]).wait()
        pltpu.make_async_copy(b_hbm.at[didx.at[pl.ds(0, n)]],
                              bufb.at[b, pl.ds(0, n)], sg[b]).wait()

    def do_add(b, n=CH3):
        @pl.loop(0, n)
        def _(r):
            for j in range(D // 16):
                sl = (b, pl.ds(r, 1), pl.ds(j * 16, 16))
                bufo.at[sl][...] = bufa.at[sl][...] + bufb.at[sl][...]

    def issue_out(c, b, n=CH3):
        pltpu.async_copy(bufo.at[b, pl.ds(0, n)],
                         out_hbm.at[pl.ds(ebase + c * CH3, n)], so[b])

    def wait_out(b, n=CH3):
        pltpu.make_async_copy(bufo.at[b, pl.ds(0, n)],
                              out_hbm.at[pl.ds(ebase, n)], so[b]).wait()

    issue_gathers(0, 0)
    issue_gathers(1, 1)

    @pl.loop(0, (F3 - 1) // 2)
    def _(p):
        for b in range(2):
            c = p * 2 + b
            wait_gathers(b)

            @pl.when(p > 0)
            def _():
                wait_out(b)

            do_add(b)
            if b == 0:
                issue_gathers(c + 2, b)
            else:
                @pl.when(p < (F3 - 1) // 2 - 1)
                def _():
                    issue_gathers(c + 2, b)
            issue_out(c, b)

    wait_gathers(0)
    wait_out(0)
    do_add(0)
    issue_out(F3 - 1, 0)
    wait_out(1)
    issue_gathers(F3, 1, T3)
    wait_gathers(1, T3)
    do_add(1, T3)
    issue_out(F3, 1, T3)
    wait_out(0)
    wait_out(1, T3)


def kernel(x, edge_index, W, b):
    src = edge_index[0].astype(jnp.int32)
    dst = edge_index[1].astype(jnp.int32)
    a = _linear_a_tc(x, W[:D], b.reshape(1, D))
    s_part = _segment_sum_sc(x, src.reshape(NW, H1, HC, CH),
                             dst.reshape(NW, H1, HC, CH))
    bb = _linear_b_tc(s_part[0], s_part[1], W[D:])
    return _edge_combine_sc(a, bb, src, dst)


# speedup vs baseline: 1.0369x; 1.0103x over previous
import functools

import jax
import jax.numpy as jnp
from jax import lax
from jax.experimental import pallas as pl
from jax.experimental.pallas import tpu as pltpu
from jax.experimental.pallas import tpu_sc as plsc

N_NODES = 10000
N_EDGES = 320000
D = 128

NC = 2
NS = 16
NW = NC * NS
EPW = N_EDGES // NW
CH = 80
CHUNKS = EPW // CH
RPT = 624
RPT_LAST = N_NODES - (NS - 1) * RPT
ZB = 8

_mesh = plsc.VectorSubcoreMesh(core_axis_name="c", subcore_axis_name="s")

H1 = 5
HC = CHUNKS // H1


@functools.partial(
    pl.kernel,
    out_type=jax.ShapeDtypeStruct((NC, N_NODES, D), jnp.float32),
    mesh=_mesh,
    scratch_types=[
        pltpu.VMEM((HC, CH), jnp.int32),
        pltpu.VMEM((HC, CH), jnp.int32),
        pltpu.VMEM((3, CH, D), jnp.float32),
        pltpu.VMEM_SHARED((N_NODES, D), jnp.float32),
        pltpu.SemaphoreType.DMA,
        pltpu.SemaphoreType.DMA,
        pltpu.SemaphoreType.DMA,
        pltpu.SemaphoreType.DMA,
        pltpu.SemaphoreType.DMA,
        pltpu.SemaphoreType.DMA,
    ],
)
def _segment_sum_sc(x_hbm, src_hbm, dst_hbm, out_hbm,
                    sidx, didx, bufx, acc,
                    sg0, sg1, sg2, ss0, ss1, ss2):
    cid = lax.axis_index("c")
    sid = lax.axis_index("s")
    wid = sid * NC + cid
    sg = (sg0, sg1, sg2)
    ss = (ss0, ss1, ss2)

    @pl.loop(0, CH)
    def _(i):
        for j in range(D // 16):
            bufx.at[0, pl.ds(i, 1), pl.ds(j * 16, 16)][...] = jnp.zeros(
                (1, 16), jnp.float32)

    @pl.when(sid < NS - 1)
    def _():
        base = sid * RPT
        for k in range(RPT // CH):
            pltpu.async_copy(bufx.at[0], acc.at[pl.ds(base + k * CH, CH)],
                             sg0)
        pltpu.async_copy(bufx.at[0, pl.ds(0, RPT % CH)],
                         acc.at[pl.ds(base + RPT - RPT % CH, RPT % CH)], sg0)
        for k in range(RPT // CH):
            pltpu.make_async_copy(bufx.at[0], acc.at[pl.ds(base, CH)],
                                  sg0).wait()
        pltpu.make_async_copy(bufx.at[0, pl.ds(0, RPT % CH)],
                              acc.at[pl.ds(base, RPT % CH)], sg0).wait()

    @pl.when(sid == NS - 1)
    def _():
        base = (NS - 1) * RPT
        for k in range(RPT_LAST // CH):
            pltpu.async_copy(bufx.at[0], acc.at[pl.ds(base + k * CH, CH)],
                             sg0)
        for k in range(RPT_LAST // CH):
            pltpu.make_async_copy(bufx.at[0], acc.at[pl.ds(base, CH)],
                                  sg0).wait()

    plsc.subcore_barrier()

    def issue_gather(c, b):
        pltpu.async_copy(x_hbm.at[sidx.at[c]], bufx.at[b], sg[b])

    def wait_gather(b):
        pltpu.make_async_copy(x_hbm.at[sidx.at[0]], bufx.at[b], sg[b]).wait()

    def issue_scatter(c, b):
        pltpu.async_copy(bufx.at[b], acc.at[didx.at[c]], ss[b], add=True)

    def wait_scatter(b):
        pltpu.make_async_copy(bufx.at[b], acc.at[didx.at[0]], ss[b]).wait()

    for h in range(H1):
        pltpu.sync_copy(src_hbm.at[wid, h], sidx)
        pltpu.sync_copy(dst_hbm.at[wid, h], didx)
        issue_gather(0, 0)
        issue_gather(1, 1)
        wait_gather(0)
        issue_scatter(0, 0)
        issue_gather(2, 2)

        @pl.loop(0, (HC - 4) // 3)
        def _(p):
            for k in range(3):
                b = (1 + k) % 3
                c = 1 + p * 3 + k
                wait_gather(b)
                issue_scatter(c, b)
                wait_scatter(k)
                issue_gather(c + 2, k)

        wait_gather((HC - 3) % 3)
        issue_scatter(HC - 3, (HC - 3) % 3)
        wait_scatter((HC - 4) % 3)
        issue_gather(HC - 1, (HC - 1) % 3)
        for c in (HC - 2, HC - 1):
            b = c % 3
            wait_gather(b)
            issue_scatter(c, b)
            wait_scatter((c - 1) % 3)
        wait_scatter((HC - 1) % 3)

    plsc.subcore_barrier()

    @pl.when(sid < NS - 1)
    def _():
        r = sid * RPT
        pltpu.sync_copy(acc.at[pl.ds(r, RPT)], out_hbm.at[cid, pl.ds(r, RPT)])

    @pl.when(sid == NS - 1)
    def _():
        r = (NS - 1) * RPT
        pltpu.sync_copy(acc.at[pl.ds(r, RPT_LAST)],
                        out_hbm.at[cid, pl.ds(r, RPT_LAST)])


_ROWS_BLK = 1000


def _linear_a_body(x_ref, w_ref, b_ref, a_ref):
    a_ref[...] = jnp.dot(x_ref[...], w_ref[...],
                         preferred_element_type=jnp.float32) + b_ref[...]


def _linear_a_tc(x, w1, bb):
    grid = (N_NODES // _ROWS_BLK,)
    blk = pl.BlockSpec((_ROWS_BLK, D), lambda i: (i, 0))
    return pl.pallas_call(
        _linear_a_body,
        grid=grid,
        in_specs=[blk,
                  pl.BlockSpec((D, D), lambda i: (0, 0)),
                  pl.BlockSpec((1, D), lambda i: (0, 0))],
        out_specs=blk,
        out_shape=jax.ShapeDtypeStruct((N_NODES, D), jnp.float32),
    )(x, w1, bb)


def _linear_b_body(s0_ref, s1_ref, w_ref, bb_ref):
    bb_ref[...] = jnp.dot(s0_ref[...] + s1_ref[...], w_ref[...],
                          preferred_element_type=jnp.float32)


def _linear_b_tc(s0, s1, w2):
    grid = (N_NODES // _ROWS_BLK,)
    blk = pl.BlockSpec((_ROWS_BLK, D), lambda i: (i, 0))
    return pl.pallas_call(
        _linear_b_body,
        grid=grid,
        in_specs=[blk, blk, pl.BlockSpec((D, D), lambda i: (0, 0))],
        out_specs=blk,
        out_shape=jax.ShapeDtypeStruct((N_NODES, D), jnp.float32),
    )(s0, s1, w2)


def _linear_body(x_ref, s0_ref, s1_ref, w_ref, b_ref, a_ref, bb_ref):
    w1 = w_ref[0:D, :]
    w2 = w_ref[D:2 * D, :]
    a_ref[...] = jnp.dot(x_ref[...], w1,
                         preferred_element_type=jnp.float32) + b_ref[...]
    bb_ref[...] = jnp.dot(s0_ref[...] + s1_ref[...], w2,
                          preferred_element_type=jnp.float32)


def _linear_tc(x, s0, s1, W, bvec):
    grid = (N_NODES // _ROWS_BLK,)
    blk = pl.BlockSpec((_ROWS_BLK, D), lambda i: (i, 0))
    return pl.pallas_call(
        _linear_body,
        grid=grid,
        in_specs=[blk, blk, blk,
                  pl.BlockSpec((2 * D, D), lambda i: (0, 0)),
                  pl.BlockSpec((1, D), lambda i: (0, 0))],
        out_specs=[blk, blk],
        out_shape=[jax.ShapeDtypeStruct((N_NODES, D), jnp.float32)] * 2,
    )(x, s0, s1, W, bvec.reshape(1, D))


CH3 = 120
F3 = EPW // CH3
T3 = EPW - F3 * CH3


@functools.partial(
    pl.kernel,
    out_type=jax.ShapeDtypeStruct((N_EDGES, D), jnp.float32),
    mesh=_mesh,
    scratch_types=[
        pltpu.VMEM((EPW,), jnp.int32),
        pltpu.VMEM((EPW,), jnp.int32),
        pltpu.VMEM((2, CH3, D), jnp.float32),
        pltpu.VMEM((2, CH3, D), jnp.float32),
        pltpu.VMEM((2, CH3, D), jnp.float32),
        pltpu.SemaphoreType.DMA,
        pltpu.SemaphoreType.DMA,
        pltpu.SemaphoreType.DMA,
        pltpu.SemaphoreType.DMA,
    ],
)
def _edge_combine_sc(a_hbm, b_hbm, src_hbm, dst_hbm, out_hbm,
                     sidx, didx, bufa, bufb, bufo, sg0, sg1, so0, so1):
    cid = lax.axis_index("c")
    sid = lax.axis_index("s")
    wid = sid * NC + cid
    ebase = wid * EPW
    sg = (sg0, sg1)
    so = (so0, so1)

    pltpu.sync_copy(src_hbm.at[pl.ds(ebase, EPW)], sidx)
    pltpu.sync_copy(dst_hbm.at[pl.ds(ebase, EPW)], didx)

    def issue_gathers(c, b, n=CH3):
        pltpu.async_copy(a_hbm.at[sidx.at[pl.ds(c * CH3, n)]],
                         bufa.at[b, pl.ds(0, n)], sg[b])
        pltpu.async_copy(b_hbm.at[didx.at[pl.ds(c * CH3, n)]],
                         bufb.at[b, pl.ds(0, n)], sg[b])

    def wait_gathers(b, n=CH3):
        pltpu.make_async_copy(a_hbm.at[sidx.at[pl.ds(0, n)]],
                              bufa.at[b, pl.ds(0, n)], sg[b]).wait()
        pltpu.make_async_copy(b_hbm.at[didx.at[pl.ds(0, n)]],
                              bufb.at[b, pl.ds(0, n)], sg[b]).wait()

    def do_add(b, n=CH3):
        @pl.loop(0, n)
        def _(r):
            for j in range(D // 16):
                sl = (b, pl.ds(r, 1), pl.ds(j * 16, 16))
                bufo.at[sl][...] = bufa.at[sl][...] + bufb.at[sl][...]

    def issue_out(c, b, n=CH3):
        pltpu.async_copy(bufo.at[b, pl.ds(0, n)],
                         out_hbm.at[pl.ds(ebase + c * CH3, n)], so[b])

    def wait_out(b, n=CH3):
        pltpu.make_async_copy(bufo.at[b, pl.ds(0, n)],
                              out_hbm.at[pl.ds(ebase, n)], so[b]).wait()

    issue_gathers(0, 0)
    issue_gathers(1, 1)

    @pl.loop(0, (F3 - 1) // 2)
    def _(p):
        for b in range(2):
            c = p * 2 + b
            wait_gathers(b)

            @pl.when(p > 0)
            def _():
                wait_out(b)

            do_add(b)
            if b == 0:
                issue_gathers(c + 2, b)
            else:
                @pl.when(p < (F3 - 1) // 2 - 1)
                def _():
                    issue_gathers(c + 2, b)
            issue_out(c, b)

    wait_gathers(0)
    wait_out(0)
    do_add(0)
    issue_out(F3 - 1, 0)
    wait_out(1)
    issue_gathers(F3, 1, T3)
    wait_gathers(1, T3)
    do_add(1, T3)
    issue_out(F3, 1, T3)
    wait_out(0)
    wait_out(1, T3)


def kernel(x, edge_index, W, b):
    src = edge_index[0].astype(jnp.int32)
    dst = edge_index[1].astype(jnp.int32)
    a = _linear_a_tc(x, W[:D], b.reshape(1, D))
    s_part = _segment_sum_sc(x, src.reshape(NW, H1, HC, CH),
                             dst.reshape(NW, H1, HC, CH))
    bb = _linear_b_tc(s_part[0], s_part[1], W[D:])
    return _edge_combine_sc(a, bb, src, dst)
